# Initial kernel scaffold; baseline (speedup 1.0000x reference)
import jax
import jax.numpy as jnp
from jax.experimental import pallas as pl


def kernel(x, adj_indices, adj_values, adj_dense, W_fc0, b_fc0, convs_W, W_fc1, b_fc1, epoch, test):
    raise NotImplementedError("write your pallas kernel here")


# speedup vs baseline: 1.0600x; 1.0600x over previous
import functools
import math

import jax
import jax.numpy as jnp
from jax.experimental import pallas as pl

N = 10000
E = 320000
NFEAT = 128
NHID = 128
NCLASS = 64
NLAYERS = 8
LAMDA = 0.5
ALPHA = 0.1

_ROW_BLK = 1000
_GRID = N // _ROW_BLK

_f32 = jnp.float32


def _fc0_body(x_ref, w_ref, b_ref, o_ref):
    h = jnp.dot(x_ref[...], w_ref[...], preferred_element_type=_f32) + b_ref[...]
    o_ref[...] = jnp.maximum(h, 0.0)


def _fc0(x, wT, b):
    return pl.pallas_call(
        _fc0_body,
        grid=(_GRID,),
        in_specs=[
            pl.BlockSpec((_ROW_BLK, NFEAT), lambda i: (i, 0)),
            pl.BlockSpec((NFEAT, NHID), lambda i: (0, 0)),
            pl.BlockSpec((1, NHID), lambda i: (0, 0)),
        ],
        out_specs=pl.BlockSpec((_ROW_BLK, NHID), lambda i: (i, 0)),
        out_shape=jax.ShapeDtypeStruct((N, NHID), _f32),
    )(x, wT, b)


def _layer_body(hi_ref, h0_ref, w_ref, o_ref, *, theta):
    s = (1.0 - ALPHA) * hi_ref[...] + ALPHA * h0_ref[...]
    out = theta * jnp.dot(s, w_ref[...], preferred_element_type=_f32) + (1.0 - theta) * s
    o_ref[...] = jnp.maximum(out, 0.0)


def _layer(hi, h0, w, theta):
    return pl.pallas_call(
        functools.partial(_layer_body, theta=theta),
        grid=(_GRID,),
        in_specs=[
            pl.BlockSpec((_ROW_BLK, NHID), lambda i: (i, 0)),
            pl.BlockSpec((_ROW_BLK, NHID), lambda i: (i, 0)),
            pl.BlockSpec((NHID, NHID), lambda i: (0, 0)),
        ],
        out_specs=pl.BlockSpec((_ROW_BLK, NHID), lambda i: (i, 0)),
        out_shape=jax.ShapeDtypeStruct((N, NHID), _f32),
    )(hi, h0, w)


def _final_body(h_ref, w_ref, b_ref, o_ref):
    logits = jnp.dot(h_ref[...], w_ref[...], preferred_element_type=_f32) + b_ref[...]
    m = jnp.max(logits, axis=1, keepdims=True)
    z = logits - m
    o_ref[...] = z - jnp.log(jnp.sum(jnp.exp(z), axis=1, keepdims=True))


def _final(h, wT, b):
    return pl.pallas_call(
        _final_body,
        grid=(_GRID,),
        in_specs=[
            pl.BlockSpec((_ROW_BLK, NHID), lambda i: (i, 0)),
            pl.BlockSpec((NHID, NCLASS), lambda i: (0, 0)),
            pl.BlockSpec((1, NCLASS), lambda i: (0, 0)),
        ],
        out_specs=pl.BlockSpec((_ROW_BLK, NCLASS), lambda i: (i, 0)),
        out_shape=jax.ShapeDtypeStruct((N, NCLASS), _f32),
    )(h, wT, b)


def kernel(x, adj_indices, adj_values, adj_dense, W_fc0, b_fc0, convs_W, W_fc1, b_fc1, epoch, test):
    row = adj_indices[0]
    col = adj_indices[1]
    h = _fc0(x, W_fc0.T, b_fc0.reshape(1, NHID))
    h0 = h
    for i in range(NLAYERS):
        theta = math.log(LAMDA / (i + 1) + 1.0)
        hi = jax.ops.segment_sum(adj_values[:, None] * h[col], row, num_segments=N)
        h = _layer(hi, h0, convs_W[i], theta)
    return _final(h, W_fc1.T, b_fc1.reshape(1, NCLASS))


# speedup vs baseline: 3.0694x; 2.8957x over previous
import functools
import math

import jax
import jax.numpy as jnp
from jax import lax
from jax.experimental import pallas as pl
from jax.experimental.pallas import tpu as pltpu
from jax.experimental.pallas import tpu_sc as plsc

N = 10000
E = 320000
NFEAT = 128
NHID = 128
NCLASS = 64
NLAYERS = 8
LAMDA = 0.5
ALPHA = 0.1

_f32 = jnp.float32

_NC = 2
_NS = 16
_NW = _NC * _NS
_L = 16
_C = 128
_NCHUNK = -(-E // (_NW * _C))
_EPW = _NCHUNK * _C
_EPAD = _NW * _EPW
_OCH = 80
_NOCH = N // _OCH
_OPS = -(-_NOCH // _NS)

_sc_mesh = plsc.VectorSubcoreMesh(core_axis_name="c", subcore_axis_name="s")


def _spmm_body(h_hbm, col_hbm, row_hbm, val_hbm, out_hbm,
               colv, rowv, valv, rows, acc, sem):
    c = lax.axis_index("c")
    s = lax.axis_index("s")
    wid = s * _NC + c

    z = jnp.zeros((_L,), _f32)

    @pl.loop(0, _C)
    def _zero_rows(j):
        for i in range(NHID // _L):
            rows[j, pl.ds(i * _L, _L)] = z

    for t in range(_OPS):
        idx = s + _NS * t

        @pl.when(idx < _NOCH)
        def _zero_acc():
            pltpu.sync_copy(rows.at[pl.ds(0, _OCH)], acc.at[pl.ds(idx * _OCH, _OCH)])
    plsc.subcore_barrier()

    ebase = wid * _EPW

    @pl.loop(0, _NCHUNK)
    def _edge_chunk(k):
        base = ebase + k * _C
        pltpu.sync_copy(col_hbm.at[pl.ds(base, _C)], colv)
        pltpu.sync_copy(row_hbm.at[pl.ds(base, _C)], rowv)
        pltpu.sync_copy(val_hbm.at[pl.ds(base, _C)], valv)
        pltpu.async_copy(h_hbm.at[colv], rows, sem).wait()

        @pl.loop(0, _C // _L)
        def _scale(g):
            vv = valv[pl.ds(g * _L, _L)]
            for j in range(_L):
                v = vv[j]
                r = g * _L + j
                for i in range(NHID // _L):
                    sl = pl.ds(i * _L, _L)
                    rows[r, sl] = rows[r, sl] * v

        pltpu.sync_copy(rows, acc.at[rowv], add=True)

    plsc.subcore_barrier()
    for t in range(_OPS):
        idx = s + _NS * t

        @pl.when(idx < _NOCH)
        def _copy_out():
            off = idx * _OCH
            pltpu.sync_copy(acc.at[pl.ds(off, _OCH)], out_hbm.at[c, pl.ds(off, _OCH)])


@functools.partial(
    pl.kernel,
    out_type=jax.ShapeDtypeStruct((_NC, N, NHID), _f32),
    mesh=_sc_mesh,
    scratch_types=[
        pltpu.VMEM((_C,), jnp.int32),
        pltpu.VMEM((_C,), jnp.int32),
        pltpu.VMEM((_C,), _f32),
        pltpu.VMEM((_C, NHID), _f32),
        pltpu.VMEM_SHARED((N, NHID), _f32),
        pltpu.SemaphoreType.DMA,
    ],
)
def _spmm(h_hbm, col_hbm, row_hbm, val_hbm, out_hbm,
          colv, rowv, valv, rows, acc, sem):
    _spmm_body(h_hbm, col_hbm, row_hbm, val_hbm, out_hbm,
               colv, rowv, valv, rows, acc, sem)


_ROW_BLK = 1000
_GRID = N // _ROW_BLK


def _fc0_body(x_ref, w_ref, b_ref, o_ref):
    h = jnp.dot(x_ref[...], w_ref[...], preferred_element_type=_f32) + b_ref[...]
    o_ref[...] = jnp.maximum(h, 0.0)


def _fc0(x, wT, b):
    return pl.pallas_call(
        _fc0_body,
        grid=(_GRID,),
        in_specs=[
            pl.BlockSpec((_ROW_BLK, NFEAT), lambda i: (i, 0)),
            pl.BlockSpec((NFEAT, NHID), lambda i: (0, 0)),
            pl.BlockSpec((1, NHID), lambda i: (0, 0)),
        ],
        out_specs=pl.BlockSpec((_ROW_BLK, NHID), lambda i: (i, 0)),
        out_shape=jax.ShapeDtypeStruct((N, NHID), _f32),
    )(x, wT, b)


def _layer_body(p0_ref, p1_ref, h0_ref, w_ref, o_ref, *, theta):
    s = (1.0 - ALPHA) * (p0_ref[...] + p1_ref[...]) + ALPHA * h0_ref[...]
    out = theta * jnp.dot(s, w_ref[...], preferred_element_type=_f32) + (1.0 - theta) * s
    o_ref[...] = jnp.maximum(out, 0.0)


def _layer(p0, p1, h0, w, theta):
    return pl.pallas_call(
        functools.partial(_layer_body, theta=theta),
        grid=(_GRID,),
        in_specs=[
            pl.BlockSpec((_ROW_BLK, NHID), lambda i: (i, 0)),
            pl.BlockSpec((_ROW_BLK, NHID), lambda i: (i, 0)),
            pl.BlockSpec((_ROW_BLK, NHID), lambda i: (i, 0)),
            pl.BlockSpec((NHID, NHID), lambda i: (0, 0)),
        ],
        out_specs=pl.BlockSpec((_ROW_BLK, NHID), lambda i: (i, 0)),
        out_shape=jax.ShapeDtypeStruct((N, NHID), _f32),
    )(p0, p1, h0, w)


def _final_body(h_ref, w_ref, b_ref, o_ref):
    logits = jnp.dot(h_ref[...], w_ref[...], preferred_element_type=_f32) + b_ref[...]
    m = jnp.max(logits, axis=1, keepdims=True)
    zc = logits - m
    o_ref[...] = zc - jnp.log(jnp.sum(jnp.exp(zc), axis=1, keepdims=True))


def _final(h, wT, b):
    return pl.pallas_call(
        _final_body,
        grid=(_GRID,),
        in_specs=[
            pl.BlockSpec((_ROW_BLK, NHID), lambda i: (i, 0)),
            pl.BlockSpec((NHID, NCLASS), lambda i: (0, 0)),
            pl.BlockSpec((1, NCLASS), lambda i: (0, 0)),
        ],
        out_specs=pl.BlockSpec((_ROW_BLK, NCLASS), lambda i: (i, 0)),
        out_shape=jax.ShapeDtypeStruct((N, NCLASS), _f32),
    )(h, wT, b)


def kernel(x, adj_indices, adj_values, adj_dense, W_fc0, b_fc0, convs_W, W_fc1, b_fc1, epoch, test):
    row = adj_indices[0]
    col = adj_indices[1]
    pad = _EPAD - E
    colp = jnp.concatenate([col, jnp.zeros((pad,), jnp.int32)])
    rowp = jnp.concatenate([row, jnp.zeros((pad,), jnp.int32)])
    valp = jnp.concatenate([adj_values, jnp.zeros((pad,), _f32)])

    h = _fc0(x, W_fc0.T, b_fc0.reshape(1, NHID))
    h0 = h
    for i in range(NLAYERS):
        theta = math.log(LAMDA / (i + 1) + 1.0)
        p = _spmm(h, colp, rowp, valp)
        h = _layer(p[0], p[1], h0, convs_W[i], theta)
    return _final(h, W_fc1.T, b_fc1.reshape(1, NCLASS))
